# baseline (speedup 1.0000x reference)
import jax
import jax.numpy as jnp
from jax.experimental import pallas as pl
from jax.experimental.pallas import tpu as pltpu

N_NODES = 100000
D = 128
E = 1024
R = 20000
NB = N_NODES // R
W = 128
EPAD = E + W


def _body(ev_ref, x_ref, wpre_ref, bpre_ref, wpost_ref, bpost_ref,
          out_ref, gsum_ref, cnt_ref, ftx_ref):
    i = pl.program_id(0)

    @pl.when(i == 0)
    def _init():
        gsum_ref[...] = jnp.zeros_like(gsum_ref)
        cnt_ref[...] = jnp.zeros_like(cnt_ref)
        ftx_ref[:, D:] = jnp.ones((R, D), jnp.bfloat16)

    xb = x_ref[0].astype(jnp.bfloat16)
    wb = wpre_ref[...].astype(jnp.bfloat16)
    pre = jax.lax.dot_general(xb, wb, (((1,), (0,)), ((), ())),
                              preferred_element_type=jnp.float32)
    ftx_ref[:, :D] = jnp.maximum(pre + bpre_ref[...], 0.0).astype(jnp.bfloat16)

    first = ev_ref[0, 0, 0]
    last = ev_ref[0, 0, R - 1]
    base = first - first % 8
    n_win = (last - base) // W + 1
    ev16 = ev_ref[0, 0, :].astype(jnp.int16)

    def _one_window(w, _):
        ws = base + w * W
        ev_rel = ev16 - ws.astype(jnp.int16)
        iota = jax.lax.broadcasted_iota(jnp.int16, (W, R), 0)
        ohb = (iota == jnp.broadcast_to(ev_rel[None, :], (W, R))
               ).astype(jnp.bfloat16)
        h = R // 8
        parts = [jax.lax.dot_general(ohb[:, j * h:(j + 1) * h],
                                     ftx_ref[j * h:(j + 1) * h, :],
                                     (((1,), (0,)), ((), ())),
                                     preferred_element_type=jnp.float32)
                 for j in range(8)]
        res = ((parts[0] + parts[1]) + (parts[2] + parts[3])) + (
            (parts[4] + parts[5]) + (parts[6] + parts[7]))
        gsum_ref[pl.ds(ws, W), :] += res[:, :D]
        cnt_ref[pl.ds(ws, W), :] += res[:, D:D + 1]
        return 0

    jax.lax.fori_loop(0, n_win, _one_window, 0)

    @pl.when(i == NB - 1)
    def _finish():
        gsum = gsum_ref[:E, :]
        gmean = gsum * (1.0 / jnp.maximum(cnt_ref[:E, :], 1.0))
        w1 = wpost_ref[:D, :]
        w2 = wpost_ref[D:, :]
        out_ref[...] = (
            jax.lax.dot_general(gsum, w1, (((1,), (0,)), ((), ())),
                                preferred_element_type=jnp.float32)
            + jax.lax.dot_general(gmean, w2, (((1,), (0,)), ((), ())),
                                  preferred_element_type=jnp.float32)
            + bpost_ref[...])


def kernel(x, event, W_pre, b_pre, W_post, b_post):
    ev = event.astype(jnp.int32)
    return pl.pallas_call(
        _body,
        grid=(NB,),
        in_specs=[
            pl.BlockSpec((1, 1, R), lambda i: (i, 0, 0)),
            pl.BlockSpec((1, R, D), lambda i: (i, 0, 0)),
            pl.BlockSpec((D, D), lambda i: (0, 0)),
            pl.BlockSpec((1, D), lambda i: (0, 0)),
            pl.BlockSpec((2 * D, D), lambda i: (0, 0)),
            pl.BlockSpec((1, D), lambda i: (0, 0)),
        ],
        out_specs=pl.BlockSpec((E, D), lambda i: (0, 0)),
        out_shape=jax.ShapeDtypeStruct((E, D), jnp.float32),
        scratch_shapes=[
            pltpu.VMEM((EPAD, D), jnp.float32),
            pltpu.VMEM((EPAD, 1), jnp.float32),
            pltpu.VMEM((R, 2 * D), jnp.bfloat16),
        ],
    )(ev.reshape(NB, 1, R), x.reshape(NB, R, D), W_pre,
      b_pre.reshape(1, D), W_post, b_post.reshape(1, D))


# speedup vs baseline: 1.0921x; 1.0921x over previous
import jax
import jax.numpy as jnp
from jax.experimental import pallas as pl
from jax.experimental.pallas import tpu as pltpu

N_NODES = 100000
D = 128
E = 1024
R = 5000
NB = N_NODES // R
W = 128
EPAD = E + W


def _body(ev_ref, x_ref, wpre_ref, bpre_ref, wpost_ref, bpost_ref,
          out_ref, gsum_ref, cnt_ref, ftx_ref):
    i = pl.program_id(0)

    @pl.when(i == 0)
    def _init():
        gsum_ref[...] = jnp.zeros_like(gsum_ref)
        cnt_ref[...] = jnp.zeros_like(cnt_ref)
        ftx_ref[:, D:] = jnp.ones((R, D), jnp.bfloat16)

    xb = x_ref[0].astype(jnp.bfloat16)
    wb = wpre_ref[...].astype(jnp.bfloat16)
    pre = jax.lax.dot_general(xb, wb, (((1,), (0,)), ((), ())),
                              preferred_element_type=jnp.float32)
    ftx_ref[:, :D] = jnp.maximum(pre + bpre_ref[...], 0.0).astype(jnp.bfloat16)

    first = ev_ref[0, 0, 0]
    last = ev_ref[0, 0, R - 1]
    base = first - first % 8
    n_win = (last - base) // W + 1
    ev16 = ev_ref[0, 0, :].astype(jnp.int16)

    def _one_window(w, _):
        ws = base + w * W
        ev_rel = ev16 - ws.astype(jnp.int16)
        iota = jax.lax.broadcasted_iota(jnp.int16, (W, R), 0)
        ohb = (iota == jnp.broadcast_to(ev_rel[None, :], (W, R))
               ).astype(jnp.bfloat16)
        h = R // 4
        parts = [jax.lax.dot_general(ohb[:, j * h:(j + 1) * h],
                                     ftx_ref[j * h:(j + 1) * h, :],
                                     (((1,), (0,)), ((), ())),
                                     preferred_element_type=jnp.float32)
                 for j in range(4)]
        res = (parts[0] + parts[1]) + (parts[2] + parts[3])
        gsum_ref[pl.ds(ws, W), :] += res[:, :D]
        cnt_ref[pl.ds(ws, W), :] += res[:, D:D + 1]
        return 0

    jax.lax.fori_loop(0, n_win, _one_window, 0)

    @pl.when(i == NB - 1)
    def _finish():
        gsum = gsum_ref[:E, :]
        gmean = gsum * (1.0 / jnp.maximum(cnt_ref[:E, :], 1.0))
        w1 = wpost_ref[:D, :]
        w2 = wpost_ref[D:, :]
        out_ref[...] = (
            jax.lax.dot_general(gsum, w1, (((1,), (0,)), ((), ())),
                                preferred_element_type=jnp.float32)
            + jax.lax.dot_general(gmean, w2, (((1,), (0,)), ((), ())),
                                  preferred_element_type=jnp.float32)
            + bpost_ref[...])


def kernel(x, event, W_pre, b_pre, W_post, b_post):
    ev = event.astype(jnp.int32)
    return pl.pallas_call(
        _body,
        grid=(NB,),
        in_specs=[
            pl.BlockSpec((1, 1, R), lambda i: (i, 0, 0)),
            pl.BlockSpec((1, R, D), lambda i: (i, 0, 0)),
            pl.BlockSpec((D, D), lambda i: (0, 0)),
            pl.BlockSpec((1, D), lambda i: (0, 0)),
            pl.BlockSpec((2 * D, D), lambda i: (0, 0)),
            pl.BlockSpec((1, D), lambda i: (0, 0)),
        ],
        out_specs=pl.BlockSpec((E, D), lambda i: (0, 0)),
        out_shape=jax.ShapeDtypeStruct((E, D), jnp.float32),
        scratch_shapes=[
            pltpu.VMEM((EPAD, D), jnp.float32),
            pltpu.VMEM((EPAD, 1), jnp.float32),
            pltpu.VMEM((R, 2 * D), jnp.bfloat16),
        ],
    )(ev.reshape(NB, 1, R), x.reshape(NB, R, D), W_pre,
      b_pre.reshape(1, D), W_post, b_post.reshape(1, D))


# speedup vs baseline: 1.2002x; 1.0989x over previous
import jax
import jax.numpy as jnp
from jax.experimental import pallas as pl
from jax.experimental.pallas import tpu as pltpu

N_NODES = 100000
D = 128
E = 1024
R = 10000
NB = N_NODES // R
W = 128
EPAD = E + W


def _body(ev_ref, x_ref, wpre_ref, bpre_ref, wpost_ref, bpost_ref,
          out_ref, gsum_ref, cnt_ref, ftx_ref):
    i = pl.program_id(0)

    @pl.when(i == 0)
    def _init():
        gsum_ref[...] = jnp.zeros_like(gsum_ref)
        cnt_ref[...] = jnp.zeros_like(cnt_ref)
        ftx_ref[:, D:] = jnp.ones((R, D), jnp.bfloat16)

    xb = x_ref[0].astype(jnp.bfloat16)
    wb = wpre_ref[...].astype(jnp.bfloat16)
    pre = jax.lax.dot_general(xb, wb, (((1,), (0,)), ((), ())),
                              preferred_element_type=jnp.float32)
    ftx_ref[:, :D] = jnp.maximum(pre + bpre_ref[...], 0.0).astype(jnp.bfloat16)

    first = ev_ref[0, 0, 0]
    last = ev_ref[0, 0, R - 1]
    base = first - first % 8
    n_win = (last - base) // W + 1
    ev16 = ev_ref[0, 0, :].astype(jnp.int16)
    iota = jax.lax.broadcasted_iota(jnp.int16, (W, R), 0)

    def _one_window(w, _):
        ws = base + w * W
        ev_rel = ev16 - ws.astype(jnp.int16)
        ohb = (iota == jnp.broadcast_to(ev_rel[None, :], (W, R))
               ).astype(jnp.bfloat16)
        h = R // 4
        parts = [jax.lax.dot_general(ohb[:, j * h:(j + 1) * h],
                                     ftx_ref[j * h:(j + 1) * h, :],
                                     (((1,), (0,)), ((), ())),
                                     preferred_element_type=jnp.float32)
                 for j in range(4)]
        res = (parts[0] + parts[1]) + (parts[2] + parts[3])
        gsum_ref[pl.ds(ws, W), :] += res[:, :D]
        cnt_ref[pl.ds(ws, W), :] += res[:, D:D + 1]
        return 0

    jax.lax.fori_loop(0, n_win, _one_window, 0)

    @pl.when(i == NB - 1)
    def _finish():
        gsum = gsum_ref[:E, :]
        gmean = gsum * (1.0 / jnp.maximum(cnt_ref[:E, :], 1.0))
        w1 = wpost_ref[:D, :]
        w2 = wpost_ref[D:, :]
        out_ref[...] = (
            jax.lax.dot_general(gsum, w1, (((1,), (0,)), ((), ())),
                                preferred_element_type=jnp.float32)
            + jax.lax.dot_general(gmean, w2, (((1,), (0,)), ((), ())),
                                  preferred_element_type=jnp.float32)
            + bpost_ref[...])


def kernel(x, event, W_pre, b_pre, W_post, b_post):
    ev = event.astype(jnp.int32)
    return pl.pallas_call(
        _body,
        grid=(NB,),
        in_specs=[
            pl.BlockSpec((1, 1, R), lambda i: (i, 0, 0)),
            pl.BlockSpec((1, R, D), lambda i: (i, 0, 0)),
            pl.BlockSpec((D, D), lambda i: (0, 0)),
            pl.BlockSpec((1, D), lambda i: (0, 0)),
            pl.BlockSpec((2 * D, D), lambda i: (0, 0)),
            pl.BlockSpec((1, D), lambda i: (0, 0)),
        ],
        out_specs=pl.BlockSpec((E, D), lambda i: (0, 0)),
        out_shape=jax.ShapeDtypeStruct((E, D), jnp.float32),
        scratch_shapes=[
            pltpu.VMEM((EPAD, D), jnp.float32),
            pltpu.VMEM((EPAD, 1), jnp.float32),
            pltpu.VMEM((R, 2 * D), jnp.bfloat16),
        ],
    )(ev.reshape(NB, 1, R), x.reshape(NB, R, D), W_pre,
      b_pre.reshape(1, D), W_post, b_post.reshape(1, D))
